# baseline (speedup 1.0000x reference)
import dataclasses
import functools

import jax
import jax.numpy as jnp
from jax import lax
from jax.experimental import pallas as pl
from jax.experimental.pallas import tpu as pltpu
from jax.experimental.pallas import tpu_sc as plsc

LANES = 16
NUM_CORES = 2
NUM_SUBCORES = 16
NUM_WORKERS = NUM_CORES * NUM_SUBCORES
ROWS_PER_BLOCK = 8
NBUF = 3


def kernel(x, permutation):
    batch, dim = x.shape
    perm = permutation.astype(jnp.int32)

    rows_per_worker = batch // NUM_WORKERS
    nblk = rows_per_worker // ROWS_PER_BLOCK
    main_blk = (nblk // NBUF) * NBUF
    tail = nblk - main_blk
    assert tail < NBUF

    mesh = plsc.VectorSubcoreMesh(core_axis_name="c", subcore_axis_name="s")

    cp = pltpu.CompilerParams()
    if "needs_layout_passes" in pltpu.CompilerParams.__dataclass_fields__:
        cp = dataclasses.replace(cp, needs_layout_passes=False)

    @functools.partial(
        pl.kernel,
        out_type=jax.ShapeDtypeStruct((batch, dim), jnp.float32),
        mesh=mesh,
        scratch_types=[
            pltpu.VMEM((dim,), jnp.int32),
            pltpu.VMEM((NBUF, ROWS_PER_BLOCK, dim), jnp.float32),
            pltpu.VMEM((NBUF, ROWS_PER_BLOCK, dim), jnp.float32),
            pltpu.SemaphoreType.DMA((NBUF,)),
            pltpu.SemaphoreType.DMA((NBUF,)),
        ],
        compiler_params=cp,
    )
    def permute_kernel(x_hbm, p_hbm, o_hbm, perm_v, inb, outb, in_sems, out_sems):
        pltpu.sync_copy(p_hbm, perm_v)

        wid = lax.axis_index("s") * NUM_CORES + lax.axis_index("c")
        row_base = wid * rows_per_worker

        row_ids = [jnp.full((LANES,), r, jnp.int32) for r in range(ROWS_PER_BLOCK)]

        def start_in(b, blk):
            src = x_hbm.at[pl.ds(row_base + blk * ROWS_PER_BLOCK, ROWS_PER_BLOCK)]
            pltpu.async_copy(src, inb.at[b], in_sems.at[b])

        def wait_in(b, blk):
            src = x_hbm.at[pl.ds(row_base + blk * ROWS_PER_BLOCK, ROWS_PER_BLOCK)]
            pltpu.make_async_copy(src, inb.at[b], in_sems.at[b]).wait()

        def start_out(b, blk):
            dst = o_hbm.at[pl.ds(row_base + blk * ROWS_PER_BLOCK, ROWS_PER_BLOCK)]
            pltpu.async_copy(outb.at[b], dst, out_sems.at[b])

        def wait_out(b, blk):
            dst = o_hbm.at[pl.ds(row_base + blk * ROWS_PER_BLOCK, ROWS_PER_BLOCK)]
            pltpu.make_async_copy(outb.at[b], dst, out_sems.at[b]).wait()

        def compute(b):
            @plsc.parallel_loop(0, dim, step=LANES, unroll=2)
            def _(j):
                idx = perm_v[pl.ds(j, LANES)]
                for r in range(ROWS_PER_BLOCK):
                    val = plsc.load_gather(inb.at[b], [row_ids[r], idx])
                    outb[b, r, pl.ds(j, LANES)] = val

        for b in range(min(NBUF, nblk)):
            start_in(b, b)

        @pl.loop(0, main_blk, step=NBUF)
        def _(i0):
            for b in range(NBUF):
                blk = i0 + b
                wait_in(b, blk)

                @pl.when(i0 > 0)
                def _():
                    wait_out(b, blk - NBUF)

                compute(b)
                start_out(b, blk)

                @pl.when(blk + NBUF < nblk)
                def _():
                    start_in(b, blk + NBUF)

        for t in range(tail):
            blk = main_blk + t
            wait_in(t, blk)
            if blk - NBUF >= 0:
                wait_out(t, blk - NBUF)
            compute(t)
            start_out(t, blk)

        for k in range(min(NBUF, nblk)):
            blk = nblk - min(NBUF, nblk) + k
            wait_out(blk % NBUF, blk)

    return permute_kernel(x, perm)


# speedup vs baseline: 1.0204x; 1.0204x over previous
import dataclasses
import functools

import jax
import jax.numpy as jnp
from jax import lax
from jax.experimental import pallas as pl
from jax.experimental.pallas import tpu as pltpu
from jax.experimental.pallas import tpu_sc as plsc

LANES = 16
NUM_CORES = 2
NUM_SUBCORES = 16
NUM_WORKERS = NUM_CORES * NUM_SUBCORES
ROWS_PER_BLOCK = 4
NBUF = 6


def kernel(x, permutation):
    batch, dim = x.shape
    perm = permutation.astype(jnp.int32)

    rows_per_worker = batch // NUM_WORKERS
    nblk = rows_per_worker // ROWS_PER_BLOCK
    main_blk = (nblk // NBUF) * NBUF
    tail = nblk - main_blk
    assert tail < NBUF

    mesh = plsc.VectorSubcoreMesh(core_axis_name="c", subcore_axis_name="s")

    cp = pltpu.CompilerParams()
    if "needs_layout_passes" in pltpu.CompilerParams.__dataclass_fields__:
        cp = dataclasses.replace(cp, needs_layout_passes=False)

    @functools.partial(
        pl.kernel,
        out_type=jax.ShapeDtypeStruct((batch, dim), jnp.float32),
        mesh=mesh,
        scratch_types=[
            pltpu.VMEM((dim,), jnp.int32),
            pltpu.VMEM((NBUF, ROWS_PER_BLOCK, dim), jnp.float32),
            pltpu.VMEM((NBUF, ROWS_PER_BLOCK, dim), jnp.float32),
            pltpu.SemaphoreType.DMA((NBUF,)),
            pltpu.SemaphoreType.DMA((NBUF,)),
            pltpu.SemaphoreType.DMA,
        ],
        compiler_params=cp,
    )
    def permute_kernel(
        x_hbm, p_hbm, o_hbm, perm_v, inb, outb, in_sems, out_sems, p_sem
    ):
        perm_copy = pltpu.async_copy(p_hbm, perm_v, p_sem)

        wid = lax.axis_index("s") * NUM_CORES + lax.axis_index("c")
        row_base = wid * rows_per_worker

        row_ids = [jnp.full((LANES,), r, jnp.int32) for r in range(ROWS_PER_BLOCK)]

        def start_in(b, blk):
            src = x_hbm.at[pl.ds(row_base + blk * ROWS_PER_BLOCK, ROWS_PER_BLOCK)]
            pltpu.async_copy(src, inb.at[b], in_sems.at[b])

        def wait_in(b, blk):
            src = x_hbm.at[pl.ds(row_base + blk * ROWS_PER_BLOCK, ROWS_PER_BLOCK)]
            pltpu.make_async_copy(src, inb.at[b], in_sems.at[b]).wait()

        def start_out(b, blk):
            dst = o_hbm.at[pl.ds(row_base + blk * ROWS_PER_BLOCK, ROWS_PER_BLOCK)]
            pltpu.async_copy(outb.at[b], dst, out_sems.at[b])

        def wait_out(b, blk):
            dst = o_hbm.at[pl.ds(row_base + blk * ROWS_PER_BLOCK, ROWS_PER_BLOCK)]
            pltpu.make_async_copy(outb.at[b], dst, out_sems.at[b]).wait()

        def compute(b):
            @plsc.parallel_loop(0, dim, step=LANES, unroll=2)
            def _(j):
                idx = perm_v[pl.ds(j, LANES)]
                for r in range(ROWS_PER_BLOCK):
                    val = plsc.load_gather(inb.at[b], [row_ids[r], idx])
                    outb[b, r, pl.ds(j, LANES)] = val

        for b in range(min(NBUF, nblk)):
            start_in(b, b)
        perm_copy.wait()

        @pl.loop(0, main_blk, step=NBUF)
        def _(i0):
            for b in range(NBUF):
                blk = i0 + b
                wait_in(b, blk)

                @pl.when(i0 > 0)
                def _():
                    wait_out(b, blk - NBUF)

                compute(b)
                start_out(b, blk)

                @pl.when(blk + NBUF < nblk)
                def _():
                    start_in(b, blk + NBUF)

        for t in range(tail):
            blk = main_blk + t
            wait_in(t, blk)
            if blk - NBUF >= 0:
                wait_out(t, blk - NBUF)
            compute(t)
            start_out(t, blk)

        for k in range(min(NBUF, nblk)):
            blk = nblk - min(NBUF, nblk) + k
            wait_out(blk % NBUF, blk)

    return permute_kernel(x, perm)
